# baseline (speedup 1.0000x reference)
import functools

import jax
import jax.numpy as jnp
from jax import lax
from jax.experimental import pallas as pl
from jax.experimental.pallas import tpu as pltpu
from jax.experimental.pallas import tpu_sc as plsc

NC = 2
NS = 16
NW = NC * NS
CH = 128
LANES = 16

SPAN = 640
N_PAD = SPAN * NS


def _sc_mesh():
  return plsc.VectorSubcoreMesh(core_axis_name="c", subcore_axis_name="s")


def _make_deg_kernel(cpw):
  @functools.partial(
      pl.kernel,
      out_type=jax.ShapeDtypeStruct((NC, N_PAD), jnp.float32),
      mesh=_sc_mesh(),
      scratch_types=[
          pltpu.VMEM((cpw, CH), jnp.int32),
          pltpu.VMEM((CH,), jnp.float32),
          pltpu.VMEM((CH,), jnp.float32),
          pltpu.VMEM_SHARED((N_PAD,), jnp.float32),
      ],
  )
  def deg_kernel(dst_hbm, out_hbm, dstv, onesv, zerov, acc):
    c = lax.axis_index("c")
    s = lax.axis_index("s")
    w = c * NS + s

    def fill(i, _):
      onesv[pl.ds(i * LANES, LANES)] = jnp.full((LANES,), 1.0, jnp.float32)
      zerov[pl.ds(i * LANES, LANES)] = jnp.zeros((LANES,), jnp.float32)
      return 0

    lax.fori_loop(0, CH // LANES, fill, 0)

    for k in range(SPAN // CH):
      pltpu.sync_copy(zerov, acc.at[pl.ds(s * SPAN + k * CH, CH)])
    plsc.subcore_barrier()

    pltpu.sync_copy(dst_hbm.at[w], dstv)

    def body(j, _):
      pltpu.sync_copy(onesv, acc.at[dstv.at[j]], add=True)
      return 0

    lax.fori_loop(0, cpw, body, 0)
    plsc.subcore_barrier()

    pltpu.sync_copy(acc.at[pl.ds(s * SPAN, SPAN)],
                    out_hbm.at[c, pl.ds(s * SPAN, SPAN)])

  return deg_kernel


def _make_prop_kernel(cpw, d):
  @functools.partial(
      pl.kernel,
      out_type=jax.ShapeDtypeStruct((NC, N_PAD, d), jnp.float32),
      mesh=_sc_mesh(),
      scratch_types=[
          pltpu.VMEM((cpw, CH), jnp.int32),
          pltpu.VMEM((cpw, CH), jnp.int32),
          pltpu.VMEM((CH, d), jnp.float32),
          pltpu.VMEM((CH, d), jnp.float32),
          pltpu.VMEM_SHARED((N_PAD, d), jnp.float32),
          pltpu.SemaphoreType.DMA,
          pltpu.SemaphoreType.DMA,
      ],
      compiler_params=pltpu.CompilerParams(use_tc_tiling_on_sc=False),
  )
  def prop_kernel(tab_hbm, src_hbm, dst_hbm, out_hbm, srcv, dstv, rows0,
                  rows1, acc, sem0, sem1):
    c = lax.axis_index("c")
    s = lax.axis_index("s")
    w = c * NS + s

    cols = d // LANES

    def fill(i, _):
      r = i // cols
      col = (i % cols) * LANES
      rows0[r, pl.ds(col, LANES)] = jnp.zeros((LANES,), jnp.float32)
      return 0

    lax.fori_loop(0, CH * cols, fill, 0)
    for k in range(SPAN // CH):
      pltpu.sync_copy(rows0, acc.at[pl.ds(s * SPAN + k * CH, CH)])
    plsc.subcore_barrier()

    pltpu.sync_copy(src_hbm.at[w], srcv)
    pltpu.sync_copy(dst_hbm.at[w], dstv)

    def body(j, _):
      pltpu.async_copy(tab_hbm.at[srcv.at[j]], rows0, sem0).wait()
      pltpu.sync_copy(rows0, acc.at[dstv.at[j]], add=True)
      return 0

    lax.fori_loop(0, cpw, body, 0)
    plsc.subcore_barrier()

    pltpu.sync_copy(acc.at[pl.ds(s * SPAN, SPAN)],
                    out_hbm.at[c, pl.ds(s * SPAN, SPAN)])

  return prop_kernel


def _row_spec(r, cols):
  return pl.BlockSpec((r, cols), lambda i: (i, 0))


def _full_spec(shape):
  return pl.BlockSpec(shape, lambda i: tuple(0 for _ in shape))


def _make_tc1(n, r, d_in, d_h):
  grid = n // r

  def body(x_ref, dega_ref, degb_ref, w1_ref, hs1_ref, dinv_ref):
    deg = dega_ref[...] + degb_ref[...] + 1.0
    dinv = 1.0 / jnp.sqrt(deg)
    h = jnp.dot(x_ref[...], w1_ref[...], preferred_element_type=jnp.float32)
    hs1_ref[...] = h * dinv
    dinv_ref[...] = dinv

  return pl.pallas_call(
      body,
      grid=(grid,),
      in_specs=[
          _row_spec(r, d_in),
          _row_spec(r, 1),
          _row_spec(r, 1),
          _full_spec((d_in, d_h)),
      ],
      out_specs=[_row_spec(r, d_h), _row_spec(r, 1)],
      out_shape=[
          jax.ShapeDtypeStruct((n, d_h), jnp.float32),
          jax.ShapeDtypeStruct((n, 1), jnp.float32),
      ],
  )


def _make_tc2(n, r, d_h, d_z):
  grid = n // r

  def body(aa_ref, ab_ref, hs1_ref, dinv_ref, b1_ref, w2_ref, hs2_ref):
    dinv = dinv_ref[...]
    h = dinv * (aa_ref[...] + ab_ref[...] + hs1_ref[...]) + b1_ref[...]
    h = jnp.maximum(h, 0.0)
    hs2_ref[...] = dinv * jnp.dot(
        h, w2_ref[...], preferred_element_type=jnp.float32)

  return pl.pallas_call(
      body,
      grid=(grid,),
      in_specs=[
          _row_spec(r, d_h),
          _row_spec(r, d_h),
          _row_spec(r, d_h),
          _row_spec(r, 1),
          _full_spec((1, d_h)),
          _full_spec((d_h, d_z)),
      ],
      out_specs=[_row_spec(r, d_z)],
      out_shape=[jax.ShapeDtypeStruct((n, d_z), jnp.float32)],
  )


def _make_tc3(n, r, d_z, d_h, d_in):
  grid = n // r

  def body(aa_ref, ab_ref, hs2_ref, dinv_ref, b2_ref, w3_ref, b3_ref, w4_ref,
           b4_ref, xhat_ref, z_ref):
    dinv = dinv_ref[...]
    z = dinv * (aa_ref[...] + ab_ref[...] + hs2_ref[...]) + b2_ref[...]
    z = jnp.maximum(z, 0.0)
    dd = jnp.dot(z, w3_ref[...], preferred_element_type=jnp.float32)
    dd = jnp.maximum(dd + b3_ref[...], 0.0)
    xhat_ref[...] = jnp.dot(
        dd, w4_ref[...], preferred_element_type=jnp.float32) + b4_ref[...]
    z_ref[...] = z

  return pl.pallas_call(
      body,
      grid=(grid,),
      in_specs=[
          _row_spec(r, d_z),
          _row_spec(r, d_z),
          _row_spec(r, d_z),
          _row_spec(r, 1),
          _full_spec((1, d_z)),
          _full_spec((d_z, d_h)),
          _full_spec((1, d_h)),
          _full_spec((d_h, d_in)),
          _full_spec((1, d_in)),
      ],
      out_specs=[_row_spec(r, d_in), _row_spec(r, d_z)],
      out_shape=[
          jax.ShapeDtypeStruct((n, d_in), jnp.float32),
          jax.ShapeDtypeStruct((n, d_z), jnp.float32),
      ],
  )


@jax.jit
def kernel(x, edge_index, W1, b1, W2, b2, W3, b3, W4, b4):
  n, d_in = x.shape
  d_h = W1.shape[1]
  d_z = W2.shape[1]
  e = edge_index.shape[1]
  assert n + 1 <= N_PAD

  cpw = -(-e // (NW * CH))
  cpw += cpw % 2
  e_pad = NW * cpw * CH
  src = edge_index[0]
  dst = edge_index[1]
  if e_pad > e:
    npad = e_pad - e
    dummy = n + jnp.arange(npad, dtype=jnp.int32) % (N_PAD - n)
    src = jnp.concatenate([src, jnp.zeros((npad,), jnp.int32)])
    dst = jnp.concatenate([dst, dummy])
  src3 = src.reshape(NW, cpw, CH)
  dst3 = dst.reshape(NW, cpw, CH)

  deg_parts = _make_deg_kernel(cpw)(dst3)
  dega = deg_parts[0, :n].reshape(n, 1)
  degb = deg_parts[1, :n].reshape(n, 1)

  r = 1000
  hs1, dinv = _make_tc1(n, r, d_in, d_h)(x, dega, degb, W1)

  acc1 = _make_prop_kernel(cpw, d_h)(hs1, src3, dst3)
  hs2, = _make_tc2(n, r, d_h, d_z)(
      acc1[0, :n], acc1[1, :n], hs1, dinv, b1.reshape(1, d_h), W2)

  acc2 = _make_prop_kernel(cpw, d_z)(hs2, src3, dst3)
  x_hat, z = _make_tc3(n, r, d_z, d_h, d_in)(
      acc2[0, :n], acc2[1, :n], hs2, dinv, b2.reshape(1, d_z), W3,
      b3.reshape(1, d_h), W4, b4.reshape(1, d_in))

  return (x_hat, z)


# speedup vs baseline: 1.1263x; 1.1263x over previous
import functools

import jax
import jax.numpy as jnp
from jax import lax
from jax.experimental import pallas as pl
from jax.experimental.pallas import tpu as pltpu
from jax.experimental.pallas import tpu_sc as plsc

NC = 2
NS = 16
NW = NC * NS
CH = 128
LANES = 16

SPAN = 640
N_PAD = SPAN * NS


def _sc_mesh():
  return plsc.VectorSubcoreMesh(core_axis_name="c", subcore_axis_name="s")


def _make_deg_kernel(cpw):
  @functools.partial(
      pl.kernel,
      out_type=jax.ShapeDtypeStruct((NC, N_PAD), jnp.float32),
      mesh=_sc_mesh(),
      scratch_types=[
          pltpu.VMEM((cpw, CH), jnp.int32),
          pltpu.VMEM((CH,), jnp.float32),
          pltpu.VMEM((CH,), jnp.float32),
          pltpu.VMEM_SHARED((N_PAD,), jnp.float32),
      ],
  )
  def deg_kernel(dst_hbm, out_hbm, dstv, onesv, zerov, acc):
    c = lax.axis_index("c")
    s = lax.axis_index("s")
    w = c * NS + s

    def fill(i, _):
      onesv[pl.ds(i * LANES, LANES)] = jnp.full((LANES,), 1.0, jnp.float32)
      zerov[pl.ds(i * LANES, LANES)] = jnp.zeros((LANES,), jnp.float32)
      return 0

    lax.fori_loop(0, CH // LANES, fill, 0)

    for k in range(SPAN // CH):
      pltpu.sync_copy(zerov, acc.at[pl.ds(s * SPAN + k * CH, CH)])
    plsc.subcore_barrier()

    pltpu.sync_copy(dst_hbm.at[w], dstv)

    def body(j, _):
      pltpu.sync_copy(onesv, acc.at[dstv.at[j]], add=True)
      return 0

    lax.fori_loop(0, cpw, body, 0)
    plsc.subcore_barrier()

    pltpu.sync_copy(acc.at[pl.ds(s * SPAN, SPAN)],
                    out_hbm.at[c, pl.ds(s * SPAN, SPAN)])

  return deg_kernel


def _make_prop_kernel(cpw, d):
  @functools.partial(
      pl.kernel,
      out_type=jax.ShapeDtypeStruct((NC, N_PAD, d), jnp.float32),
      mesh=_sc_mesh(),
      scratch_types=[
          pltpu.VMEM((cpw, CH), jnp.int32),
          pltpu.VMEM((cpw, CH), jnp.int32),
          pltpu.VMEM((CH, d), jnp.float32),
          pltpu.VMEM((CH, d), jnp.float32),
          pltpu.VMEM_SHARED((N_PAD, d), jnp.float32),
          pltpu.SemaphoreType.DMA,
          pltpu.SemaphoreType.DMA,
      ],
      compiler_params=pltpu.CompilerParams(use_tc_tiling_on_sc=False),
  )
  def prop_kernel(tab_hbm, src_hbm, dst_hbm, out_hbm, srcv, dstv, rows0,
                  rows1, acc, sem0, sem1):
    c = lax.axis_index("c")
    s = lax.axis_index("s")
    w = c * NS + s

    cols = d // LANES

    def fill(i, _):
      r = i // cols
      col = (i % cols) * LANES
      rows0[r, pl.ds(col, LANES)] = jnp.zeros((LANES,), jnp.float32)
      return 0

    lax.fori_loop(0, CH * cols, fill, 0)
    for k in range(SPAN // CH):
      pltpu.sync_copy(rows0, acc.at[pl.ds(s * SPAN + k * CH, CH)])
    plsc.subcore_barrier()

    pltpu.sync_copy(src_hbm.at[w], srcv)
    pltpu.sync_copy(dst_hbm.at[w], dstv)

    def gather(j, rbuf, sm):
      return pltpu.make_async_copy(tab_hbm.at[srcv.at[j]], rbuf, sm)

    gather(0, rows0, sem0).start()
    gather(1, rows1, sem1).start()

    def body(i, _):
      j0 = 2 * i
      gather(j0, rows0, sem0).wait()
      pltpu.sync_copy(rows0, acc.at[dstv.at[j0]], add=True)
      gather(j0 + 2, rows0, sem0).start()
      gather(j0 + 1, rows1, sem1).wait()
      pltpu.sync_copy(rows1, acc.at[dstv.at[j0 + 1]], add=True)
      gather(j0 + 3, rows1, sem1).start()
      return 0

    lax.fori_loop(0, cpw // 2 - 1, body, 0)
    jlast = cpw - 2
    gather(jlast, rows0, sem0).wait()
    pltpu.sync_copy(rows0, acc.at[dstv.at[jlast]], add=True)
    gather(jlast + 1, rows1, sem1).wait()
    pltpu.sync_copy(rows1, acc.at[dstv.at[jlast + 1]], add=True)
    plsc.subcore_barrier()

    pltpu.sync_copy(acc.at[pl.ds(s * SPAN, SPAN)],
                    out_hbm.at[c, pl.ds(s * SPAN, SPAN)])

  return prop_kernel


def _row_spec(r, cols):
  return pl.BlockSpec((r, cols), lambda i: (i, 0))


def _full_spec(shape):
  return pl.BlockSpec(shape, lambda i: tuple(0 for _ in shape))


def _make_tc1(n, r, d_in, d_h):
  grid = n // r

  def body(x_ref, dega_ref, degb_ref, w1_ref, hs1_ref, dinv_ref):
    deg = dega_ref[...] + degb_ref[...] + 1.0
    dinv = 1.0 / jnp.sqrt(deg)
    h = jnp.dot(x_ref[...], w1_ref[...], preferred_element_type=jnp.float32)
    hs1_ref[...] = h * dinv
    dinv_ref[...] = dinv

  return pl.pallas_call(
      body,
      grid=(grid,),
      in_specs=[
          _row_spec(r, d_in),
          _row_spec(r, 1),
          _row_spec(r, 1),
          _full_spec((d_in, d_h)),
      ],
      out_specs=[_row_spec(r, d_h), _row_spec(r, 1)],
      out_shape=[
          jax.ShapeDtypeStruct((n, d_h), jnp.float32),
          jax.ShapeDtypeStruct((n, 1), jnp.float32),
      ],
  )


def _make_tc2(n, r, d_h, d_z):
  grid = n // r

  def body(aa_ref, ab_ref, hs1_ref, dinv_ref, b1_ref, w2_ref, hs2_ref):
    dinv = dinv_ref[...]
    h = dinv * (aa_ref[...] + ab_ref[...] + hs1_ref[...]) + b1_ref[...]
    h = jnp.maximum(h, 0.0)
    hs2_ref[...] = dinv * jnp.dot(
        h, w2_ref[...], preferred_element_type=jnp.float32)

  return pl.pallas_call(
      body,
      grid=(grid,),
      in_specs=[
          _row_spec(r, d_h),
          _row_spec(r, d_h),
          _row_spec(r, d_h),
          _row_spec(r, 1),
          _full_spec((1, d_h)),
          _full_spec((d_h, d_z)),
      ],
      out_specs=[_row_spec(r, d_z)],
      out_shape=[jax.ShapeDtypeStruct((n, d_z), jnp.float32)],
  )


def _make_tc3(n, r, d_z, d_h, d_in):
  grid = n // r

  def body(aa_ref, ab_ref, hs2_ref, dinv_ref, b2_ref, w3_ref, b3_ref, w4_ref,
           b4_ref, xhat_ref, z_ref):
    dinv = dinv_ref[...]
    z = dinv * (aa_ref[...] + ab_ref[...] + hs2_ref[...]) + b2_ref[...]
    z = jnp.maximum(z, 0.0)
    dd = jnp.dot(z, w3_ref[...], preferred_element_type=jnp.float32)
    dd = jnp.maximum(dd + b3_ref[...], 0.0)
    xhat_ref[...] = jnp.dot(
        dd, w4_ref[...], preferred_element_type=jnp.float32) + b4_ref[...]
    z_ref[...] = z

  return pl.pallas_call(
      body,
      grid=(grid,),
      in_specs=[
          _row_spec(r, d_z),
          _row_spec(r, d_z),
          _row_spec(r, d_z),
          _row_spec(r, 1),
          _full_spec((1, d_z)),
          _full_spec((d_z, d_h)),
          _full_spec((1, d_h)),
          _full_spec((d_h, d_in)),
          _full_spec((1, d_in)),
      ],
      out_specs=[_row_spec(r, d_in), _row_spec(r, d_z)],
      out_shape=[
          jax.ShapeDtypeStruct((n, d_in), jnp.float32),
          jax.ShapeDtypeStruct((n, d_z), jnp.float32),
      ],
  )


@jax.jit
def kernel(x, edge_index, W1, b1, W2, b2, W3, b3, W4, b4):
  n, d_in = x.shape
  d_h = W1.shape[1]
  d_z = W2.shape[1]
  e = edge_index.shape[1]
  assert n + 1 <= N_PAD

  cpw = -(-e // (NW * CH))
  cpw += cpw % 2
  e_pad = NW * cpw * CH
  src = edge_index[0]
  dst = edge_index[1]
  if e_pad > e:
    npad = e_pad - e
    src = jnp.concatenate([src, jnp.zeros((npad,), jnp.int32)])
    dst = jnp.concatenate([dst, jnp.full((npad,), n, jnp.int32)])
  src3 = src.reshape(NW, cpw, CH)
  dst3 = dst.reshape(NW, cpw, CH)

  deg_parts = _make_deg_kernel(cpw)(dst3)
  dega = deg_parts[0, :n].reshape(n, 1)
  degb = deg_parts[1, :n].reshape(n, 1)

  r = 1000
  hs1, dinv = _make_tc1(n, r, d_in, d_h)(x, dega, degb, W1)

  acc1 = _make_prop_kernel(cpw, d_h)(hs1, src3, dst3)
  hs2, = _make_tc2(n, r, d_h, d_z)(
      acc1[0, :n], acc1[1, :n], hs1, dinv, b1.reshape(1, d_h), W2)

  acc2 = _make_prop_kernel(cpw, d_z)(hs2, src3, dst3)
  x_hat, z = _make_tc3(n, r, d_z, d_h, d_in)(
      acc2[0, :n], acc2[1, :n], hs2, dinv, b2.reshape(1, d_z), W3,
      b3.reshape(1, d_h), W4, b4.reshape(1, d_in))

  return (x_hat, z)


# speedup vs baseline: 1.3160x; 1.1684x over previous
import functools

import jax
import jax.numpy as jnp
from jax import lax
from jax.experimental import pallas as pl
from jax.experimental.pallas import tpu as pltpu
from jax.experimental.pallas import tpu_sc as plsc

NC = 2
NS = 16
NW = NC * NS
CH = 128
LANES = 16

SPAN = 640
N_PAD = SPAN * NS


def _sc_mesh():
  return plsc.VectorSubcoreMesh(core_axis_name="c", subcore_axis_name="s")


def _make_deg_kernel(cpw):
  @functools.partial(
      pl.kernel,
      out_type=jax.ShapeDtypeStruct((NC, N_PAD), jnp.float32),
      mesh=_sc_mesh(),
      scratch_types=[
          pltpu.VMEM((cpw, CH), jnp.int32),
          pltpu.VMEM((CH,), jnp.float32),
          pltpu.VMEM((CH,), jnp.float32),
          pltpu.VMEM_SHARED((N_PAD,), jnp.float32),
      ],
  )
  def deg_kernel(dst_hbm, out_hbm, dstv, onesv, zerov, acc):
    c = lax.axis_index("c")
    s = lax.axis_index("s")
    w = s * NC + c

    def fill(i, _):
      onesv[pl.ds(i * LANES, LANES)] = jnp.full((LANES,), 1.0, jnp.float32)
      zerov[pl.ds(i * LANES, LANES)] = jnp.zeros((LANES,), jnp.float32)
      return 0

    lax.fori_loop(0, CH // LANES, fill, 0)

    for k in range(SPAN // CH):
      pltpu.sync_copy(zerov, acc.at[pl.ds(s * SPAN + k * CH, CH)])
    plsc.subcore_barrier()

    pltpu.sync_copy(dst_hbm.at[w], dstv)

    def body(j, _):
      pltpu.sync_copy(onesv, acc.at[dstv.at[j]], add=True)
      return 0

    lax.fori_loop(0, cpw, body, 0)
    plsc.subcore_barrier()

    pltpu.sync_copy(acc.at[pl.ds(s * SPAN, SPAN)],
                    out_hbm.at[c, pl.ds(s * SPAN, SPAN)])

  return deg_kernel


def _make_prop_kernel(cpw, d):
  @functools.partial(
      pl.kernel,
      out_type=jax.ShapeDtypeStruct((NC, N_PAD, d), jnp.float32),
      mesh=_sc_mesh(),
      scratch_types=[
          pltpu.VMEM((cpw, CH), jnp.int32),
          pltpu.VMEM((cpw, CH), jnp.int32),
          pltpu.VMEM((CH, d), jnp.float32),
          pltpu.VMEM((CH, d), jnp.float32),
          pltpu.VMEM_SHARED((N_PAD, d), jnp.float32),
          pltpu.SemaphoreType.DMA,
          pltpu.SemaphoreType.DMA,
      ],
      compiler_params=pltpu.CompilerParams(use_tc_tiling_on_sc=False),
  )
  def prop_kernel(tab_hbm, src_hbm, dst_hbm, out_hbm, srcv, dstv, rows0,
                  rows1, acc, sem0, sem1):
    c = lax.axis_index("c")
    s = lax.axis_index("s")
    w = s * NC + c

    cols = d // LANES

    def fill(i, _):
      r = i // cols
      col = (i % cols) * LANES
      rows0[r, pl.ds(col, LANES)] = jnp.zeros((LANES,), jnp.float32)
      return 0

    lax.fori_loop(0, CH * cols, fill, 0)
    for k in range(SPAN // CH):
      pltpu.sync_copy(rows0, acc.at[pl.ds(s * SPAN + k * CH, CH)])
    plsc.subcore_barrier()

    pltpu.sync_copy(src_hbm.at[w], srcv)
    pltpu.sync_copy(dst_hbm.at[w], dstv)

    def body(j, _):
      pltpu.async_copy(tab_hbm.at[srcv.at[j]], rows0, sem0).wait()
      pltpu.sync_copy(rows0, acc.at[dstv.at[j]], add=True)
      return 0

    lax.fori_loop(0, cpw, body, 0)
    plsc.subcore_barrier()

    pltpu.sync_copy(acc.at[pl.ds(s * SPAN, SPAN)],
                    out_hbm.at[c, pl.ds(s * SPAN, SPAN)])

  return prop_kernel


def _row_spec(r, cols):
  return pl.BlockSpec((r, cols), lambda i: (i, 0))


def _full_spec(shape):
  return pl.BlockSpec(shape, lambda i: tuple(0 for _ in shape))


def _make_tc1(n, r, d_in, d_h):
  grid = n // r

  def body(x_ref, dega_ref, degb_ref, w1_ref, hs1_ref, dinv_ref):
    deg = dega_ref[...] + degb_ref[...] + 1.0
    dinv = 1.0 / jnp.sqrt(deg)
    h = jnp.dot(x_ref[...], w1_ref[...], preferred_element_type=jnp.float32)
    hs1_ref[...] = h * dinv
    dinv_ref[...] = dinv

  return pl.pallas_call(
      body,
      grid=(grid,),
      in_specs=[
          _row_spec(r, d_in),
          _row_spec(r, 1),
          _row_spec(r, 1),
          _full_spec((d_in, d_h)),
      ],
      out_specs=[_row_spec(r, d_h), _row_spec(r, 1)],
      out_shape=[
          jax.ShapeDtypeStruct((n, d_h), jnp.float32),
          jax.ShapeDtypeStruct((n, 1), jnp.float32),
      ],
  )


def _make_tc2(n, r, d_h, d_z):
  grid = n // r

  def body(aa_ref, ab_ref, hs1_ref, dinv_ref, b1_ref, w2_ref, hs2_ref):
    dinv = dinv_ref[...]
    h = dinv * (aa_ref[...] + ab_ref[...] + hs1_ref[...]) + b1_ref[...]
    h = jnp.maximum(h, 0.0)
    hs2_ref[...] = dinv * jnp.dot(
        h, w2_ref[...], preferred_element_type=jnp.float32)

  return pl.pallas_call(
      body,
      grid=(grid,),
      in_specs=[
          _row_spec(r, d_h),
          _row_spec(r, d_h),
          _row_spec(r, d_h),
          _row_spec(r, 1),
          _full_spec((1, d_h)),
          _full_spec((d_h, d_z)),
      ],
      out_specs=[_row_spec(r, d_z)],
      out_shape=[jax.ShapeDtypeStruct((n, d_z), jnp.float32)],
  )


def _make_tc3(n, r, d_z, d_h, d_in):
  grid = n // r

  def body(aa_ref, ab_ref, hs2_ref, dinv_ref, b2_ref, w3_ref, b3_ref, w4_ref,
           b4_ref, xhat_ref, z_ref):
    dinv = dinv_ref[...]
    z = dinv * (aa_ref[...] + ab_ref[...] + hs2_ref[...]) + b2_ref[...]
    z = jnp.maximum(z, 0.0)
    dd = jnp.dot(z, w3_ref[...], preferred_element_type=jnp.float32)
    dd = jnp.maximum(dd + b3_ref[...], 0.0)
    xhat_ref[...] = jnp.dot(
        dd, w4_ref[...], preferred_element_type=jnp.float32) + b4_ref[...]
    z_ref[...] = z

  return pl.pallas_call(
      body,
      grid=(grid,),
      in_specs=[
          _row_spec(r, d_z),
          _row_spec(r, d_z),
          _row_spec(r, d_z),
          _row_spec(r, 1),
          _full_spec((1, d_z)),
          _full_spec((d_z, d_h)),
          _full_spec((1, d_h)),
          _full_spec((d_h, d_in)),
          _full_spec((1, d_in)),
      ],
      out_specs=[_row_spec(r, d_in), _row_spec(r, d_z)],
      out_shape=[
          jax.ShapeDtypeStruct((n, d_in), jnp.float32),
          jax.ShapeDtypeStruct((n, d_z), jnp.float32),
      ],
  )


@jax.jit
def kernel(x, edge_index, W1, b1, W2, b2, W3, b3, W4, b4):
  n, d_in = x.shape
  d_h = W1.shape[1]
  d_z = W2.shape[1]
  e = edge_index.shape[1]
  assert n + 1 <= N_PAD

  cpw = -(-e // (NW * CH))
  e_pad = NW * cpw * CH
  src = edge_index[0]
  dst = edge_index[1]
  if e_pad > e:
    npad = e_pad - e
    src = jnp.concatenate([src, jnp.zeros((npad,), jnp.int32)])
    dst = jnp.concatenate([dst, jnp.full((npad,), n, jnp.int32)])
  src3 = src.reshape(NW, cpw, CH)
  dst3 = dst.reshape(NW, cpw, CH)

  deg_parts = _make_deg_kernel(cpw)(dst3)
  dega = deg_parts[0, :n].reshape(n, 1)
  degb = deg_parts[1, :n].reshape(n, 1)

  r = 1000
  hs1, dinv = _make_tc1(n, r, d_in, d_h)(x, dega, degb, W1)

  acc1 = _make_prop_kernel(cpw, d_h)(hs1, src3, dst3)
  hs2, = _make_tc2(n, r, d_h, d_z)(
      acc1[0, :n], acc1[1, :n], hs1, dinv, b1.reshape(1, d_h), W2)

  acc2 = _make_prop_kernel(cpw, d_z)(hs2, src3, dst3)
  x_hat, z = _make_tc3(n, r, d_z, d_h, d_in)(
      acc2[0, :n], acc2[1, :n], hs2, dinv, b2.reshape(1, d_z), W3,
      b3.reshape(1, d_h), W4, b4.reshape(1, d_in))

  return (x_hat, z)


# speedup vs baseline: 1.3227x; 1.0051x over previous
import functools

import jax
import jax.numpy as jnp
from jax import lax
from jax.experimental import pallas as pl
from jax.experimental.pallas import tpu as pltpu
from jax.experimental.pallas import tpu_sc as plsc

NC = 2
NS = 16
NW = NC * NS
CH = 128
LANES = 16

SPAN = 640
N_PAD = SPAN * NS


def _sc_mesh():
  return plsc.VectorSubcoreMesh(core_axis_name="c", subcore_axis_name="s")


def _make_deg_kernel(cpw):
  @functools.partial(
      pl.kernel,
      out_type=jax.ShapeDtypeStruct((NC, N_PAD), jnp.float32),
      mesh=_sc_mesh(),
      scratch_types=[
          pltpu.VMEM((cpw, CH), jnp.int32),
          pltpu.VMEM((CH,), jnp.float32),
          pltpu.VMEM((CH,), jnp.float32),
          pltpu.VMEM_SHARED((N_PAD,), jnp.float32),
      ],
  )
  def deg_kernel(dst_hbm, out_hbm, dstv, onesv, zerov, acc):
    c = lax.axis_index("c")
    s = lax.axis_index("s")
    w = s * NC + c

    def fill(i, _):
      onesv[pl.ds(i * LANES, LANES)] = jnp.full((LANES,), 1.0, jnp.float32)
      zerov[pl.ds(i * LANES, LANES)] = jnp.zeros((LANES,), jnp.float32)
      return 0

    lax.fori_loop(0, CH // LANES, fill, 0)

    for k in range(SPAN // CH):
      pltpu.sync_copy(zerov, acc.at[pl.ds(s * SPAN + k * CH, CH)])
    plsc.subcore_barrier()

    pltpu.sync_copy(dst_hbm.at[w], dstv)

    def body(j, _):
      pltpu.sync_copy(onesv, acc.at[dstv.at[j]], add=True)
      return 0

    lax.fori_loop(0, cpw, body, 0)
    plsc.subcore_barrier()

    pltpu.sync_copy(acc.at[pl.ds(s * SPAN, SPAN)],
                    out_hbm.at[c, pl.ds(s * SPAN, SPAN)])

  return deg_kernel


def _make_prop_kernel(cpw, d):
  @functools.partial(
      pl.kernel,
      out_type=jax.ShapeDtypeStruct((NC, N_PAD, d), jnp.float32),
      mesh=_sc_mesh(),
      scratch_types=[
          pltpu.VMEM((cpw, CH), jnp.int32),
          pltpu.VMEM((cpw, CH), jnp.int32),
          pltpu.VMEM((CH, d), jnp.float32),
          pltpu.VMEM((CH, d), jnp.float32),
          pltpu.VMEM_SHARED((N_PAD, d), jnp.float32),
          pltpu.SemaphoreType.DMA,
          pltpu.SemaphoreType.DMA,
      ],
      compiler_params=pltpu.CompilerParams(use_tc_tiling_on_sc=False),
  )
  def prop_kernel(tab_hbm, src_hbm, dst_hbm, out_hbm, srcv, dstv, rows0,
                  rows1, acc, sem0, sem1):
    c = lax.axis_index("c")
    s = lax.axis_index("s")
    w = s * NC + c

    cols = d // LANES

    def fill(i, _):
      r = i // cols
      col = (i % cols) * LANES
      rows0[r, pl.ds(col, LANES)] = jnp.zeros((LANES,), jnp.float32)
      return 0

    lax.fori_loop(0, CH * cols, fill, 0)
    for k in range(SPAN // CH):
      pltpu.sync_copy(rows0, acc.at[pl.ds(s * SPAN + k * CH, CH)])
    plsc.subcore_barrier()

    pltpu.sync_copy(src_hbm.at[w], srcv)
    pltpu.sync_copy(dst_hbm.at[w], dstv)

    def body(j, _):
      pltpu.async_copy(tab_hbm.at[srcv.at[j]], rows0, sem0).wait()
      pltpu.sync_copy(rows0, acc.at[dstv.at[j]], add=True)
      return 0

    lax.fori_loop(0, cpw, body, 0)
    plsc.subcore_barrier()

    pltpu.sync_copy(acc.at[pl.ds(s * SPAN, SPAN)],
                    out_hbm.at[c, pl.ds(s * SPAN, SPAN)])

  return prop_kernel


def _row_spec(r, cols):
  return pl.BlockSpec((r, cols), lambda i: (i, 0))


def _full_spec(shape):
  return pl.BlockSpec(shape, lambda i: tuple(0 for _ in shape))


def _make_tc1(n, r, d_in, d_h):
  grid = n // r

  def body(x_ref, dega_ref, degb_ref, w1_ref, hs1_ref, dinv_ref):
    deg = dega_ref[...] + degb_ref[...] + 1.0
    dinv = 1.0 / jnp.sqrt(deg)
    h = jnp.dot(x_ref[...], w1_ref[...], preferred_element_type=jnp.float32)
    hs1_ref[...] = h * dinv
    dinv_ref[...] = dinv

  return pl.pallas_call(
      body,
      grid=(grid,),
      in_specs=[
          _row_spec(r, d_in),
          _row_spec(r, 1),
          _row_spec(r, 1),
          _full_spec((d_in, d_h)),
      ],
      out_specs=[_row_spec(r, d_h), _row_spec(r, 1)],
      out_shape=[
          jax.ShapeDtypeStruct((n, d_h), jnp.float32),
          jax.ShapeDtypeStruct((n, 1), jnp.float32),
      ],
  )


def _make_tc2(n, r, d_h, d_z):
  grid = n // r

  def body(aa_ref, ab_ref, hs1_ref, dinv_ref, b1_ref, w2_ref, hs2_ref):
    dinv = dinv_ref[...]
    h = dinv * (aa_ref[...] + ab_ref[...] + hs1_ref[...]) + b1_ref[...]
    h = jnp.maximum(h, 0.0)
    hs2_ref[...] = dinv * jnp.dot(
        h, w2_ref[...], preferred_element_type=jnp.float32)

  return pl.pallas_call(
      body,
      grid=(grid,),
      in_specs=[
          _row_spec(r, d_h),
          _row_spec(r, d_h),
          _row_spec(r, d_h),
          _row_spec(r, 1),
          _full_spec((1, d_h)),
          _full_spec((d_h, d_z)),
      ],
      out_specs=[_row_spec(r, d_z)],
      out_shape=[jax.ShapeDtypeStruct((n, d_z), jnp.float32)],
  )


def _make_tc3(n, r, d_z, d_h, d_in):
  grid = n // r

  def body(aa_ref, ab_ref, hs2_ref, dinv_ref, b2_ref, w3_ref, b3_ref, w4_ref,
           b4_ref, xhat_ref, z_ref):
    dinv = dinv_ref[...]
    z = dinv * (aa_ref[...] + ab_ref[...] + hs2_ref[...]) + b2_ref[...]
    z = jnp.maximum(z, 0.0)
    dd = jnp.dot(z, w3_ref[...], preferred_element_type=jnp.float32)
    dd = jnp.maximum(dd + b3_ref[...], 0.0)
    xhat_ref[...] = jnp.dot(
        dd, w4_ref[...], preferred_element_type=jnp.float32) + b4_ref[...]
    z_ref[...] = z

  return pl.pallas_call(
      body,
      grid=(grid,),
      in_specs=[
          _row_spec(r, d_z),
          _row_spec(r, d_z),
          _row_spec(r, d_z),
          _row_spec(r, 1),
          _full_spec((1, d_z)),
          _full_spec((d_z, d_h)),
          _full_spec((1, d_h)),
          _full_spec((d_h, d_in)),
          _full_spec((1, d_in)),
      ],
      out_specs=[_row_spec(r, d_in), _row_spec(r, d_z)],
      out_shape=[
          jax.ShapeDtypeStruct((n, d_in), jnp.float32),
          jax.ShapeDtypeStruct((n, d_z), jnp.float32),
      ],
  )


@jax.jit
def kernel(x, edge_index, W1, b1, W2, b2, W3, b3, W4, b4):
  n, d_in = x.shape
  d_h = W1.shape[1]
  d_z = W2.shape[1]
  e = edge_index.shape[1]
  assert n + 1 <= N_PAD

  cpw = -(-e // (NW * CH))
  e_pad = NW * cpw * CH
  src = edge_index[0]
  dst = edge_index[1]
  if e_pad > e:
    npad = e_pad - e
    dummy = n + jnp.arange(npad, dtype=jnp.int32) % (N_PAD - n)
    src = jnp.concatenate([src, jnp.zeros((npad,), jnp.int32)])
    dst = jnp.concatenate([dst, dummy])
  src3 = src.reshape(NW, cpw, CH)
  dst3 = dst.reshape(NW, cpw, CH)

  deg_parts = _make_deg_kernel(cpw)(dst3)
  dega = deg_parts[0, :n].reshape(n, 1)
  degb = deg_parts[1, :n].reshape(n, 1)

  r = 1000
  hs1, dinv = _make_tc1(n, r, d_in, d_h)(x, dega, degb, W1)

  acc1 = _make_prop_kernel(cpw, d_h)(hs1, src3, dst3)
  hs2, = _make_tc2(n, r, d_h, d_z)(
      acc1[0, :n], acc1[1, :n], hs1, dinv, b1.reshape(1, d_h), W2)

  acc2 = _make_prop_kernel(cpw, d_z)(hs2, src3, dst3)
  x_hat, z = _make_tc3(n, r, d_z, d_h, d_in)(
      acc2[0, :n], acc2[1, :n], hs2, dinv, b2.reshape(1, d_z), W3,
      b3.reshape(1, d_h), W4, b4.reshape(1, d_in))

  return (x_hat, z)


# speedup vs baseline: 1.5009x; 1.1347x over previous
import functools

import jax
import jax.numpy as jnp
from jax import lax
from jax.experimental import pallas as pl
from jax.experimental.pallas import tpu as pltpu
from jax.experimental.pallas import tpu_sc as plsc

NC = 2
NS = 16
NW = NC * NS
CH = 80
LANES = 16

SPAN = 640
N_PAD = SPAN * NS


def _sc_mesh():
  return plsc.VectorSubcoreMesh(core_axis_name="c", subcore_axis_name="s")


def _make_deg_kernel(cpw):
  @functools.partial(
      pl.kernel,
      out_type=jax.ShapeDtypeStruct((NC, N_PAD), jnp.float32),
      mesh=_sc_mesh(),
      scratch_types=[
          pltpu.VMEM((cpw, CH), jnp.int32),
          pltpu.VMEM((CH,), jnp.float32),
          pltpu.VMEM((CH,), jnp.float32),
          pltpu.VMEM_SHARED((N_PAD,), jnp.float32),
      ],
  )
  def deg_kernel(dst_hbm, out_hbm, dstv, onesv, zerov, acc):
    c = lax.axis_index("c")
    s = lax.axis_index("s")
    w = s * NC + c

    def fill(i, _):
      onesv[pl.ds(i * LANES, LANES)] = jnp.full((LANES,), 1.0, jnp.float32)
      zerov[pl.ds(i * LANES, LANES)] = jnp.zeros((LANES,), jnp.float32)
      return 0

    lax.fori_loop(0, CH // LANES, fill, 0)

    for k in range(SPAN // CH):
      pltpu.sync_copy(zerov, acc.at[pl.ds(s * SPAN + k * CH, CH)])
    plsc.subcore_barrier()

    pltpu.sync_copy(dst_hbm.at[w], dstv)

    def body(j, _):
      pltpu.sync_copy(onesv, acc.at[dstv.at[j]], add=True)
      return 0

    lax.fori_loop(0, cpw, body, 0)
    plsc.subcore_barrier()

    pltpu.sync_copy(acc.at[pl.ds(s * SPAN, SPAN)],
                    out_hbm.at[c, pl.ds(s * SPAN, SPAN)])

  return deg_kernel


def _make_prop_kernel(cpw, d):
  @functools.partial(
      pl.kernel,
      out_type=jax.ShapeDtypeStruct((NC, N_PAD, d), jnp.float32),
      mesh=_sc_mesh(),
      scratch_types=[
          pltpu.VMEM((cpw, CH), jnp.int32),
          pltpu.VMEM((cpw, CH), jnp.int32),
          pltpu.VMEM((CH, d), jnp.float32),
          pltpu.VMEM((CH, d), jnp.float32),
          pltpu.VMEM_SHARED((N_PAD, d), jnp.float32),
          pltpu.SemaphoreType.DMA,
          pltpu.SemaphoreType.DMA,
      ],
      compiler_params=pltpu.CompilerParams(use_tc_tiling_on_sc=False),
  )
  def prop_kernel(tab_hbm, src_hbm, dst_hbm, out_hbm, srcv, dstv, rows0,
                  rows1, acc, sem0, sem1):
    c = lax.axis_index("c")
    s = lax.axis_index("s")
    w = s * NC + c

    cols = d // LANES

    def fill(i, _):
      r = i // cols
      col = (i % cols) * LANES
      rows0[r, pl.ds(col, LANES)] = jnp.zeros((LANES,), jnp.float32)
      return 0

    lax.fori_loop(0, CH * cols, fill, 0)
    for k in range(SPAN // CH):
      pltpu.sync_copy(rows0, acc.at[pl.ds(s * SPAN + k * CH, CH)])
    plsc.subcore_barrier()

    pltpu.sync_copy(src_hbm.at[w], srcv)
    pltpu.sync_copy(dst_hbm.at[w], dstv)

    def body(j, _):
      pltpu.async_copy(tab_hbm.at[srcv.at[j]], rows0, sem0).wait()
      pltpu.sync_copy(rows0, acc.at[dstv.at[j]], add=True)
      return 0

    lax.fori_loop(0, cpw, body, 0)
    plsc.subcore_barrier()

    pltpu.sync_copy(acc.at[pl.ds(s * SPAN, SPAN)],
                    out_hbm.at[c, pl.ds(s * SPAN, SPAN)])

  return prop_kernel


def _row_spec(r, cols):
  return pl.BlockSpec((r, cols), lambda i: (i, 0))


def _full_spec(shape):
  return pl.BlockSpec(shape, lambda i: tuple(0 for _ in shape))


def _make_tc1(n, r, d_in, d_h):
  grid = n // r

  def body(x_ref, dega_ref, degb_ref, w1_ref, hs1_ref, dinv_ref):
    deg = dega_ref[...] + degb_ref[...] + 1.0
    dinv = 1.0 / jnp.sqrt(deg)
    h = jnp.dot(x_ref[...], w1_ref[...], preferred_element_type=jnp.float32)
    hs1_ref[...] = h * dinv
    dinv_ref[...] = dinv

  return pl.pallas_call(
      body,
      grid=(grid,),
      in_specs=[
          _row_spec(r, d_in),
          _row_spec(r, 1),
          _row_spec(r, 1),
          _full_spec((d_in, d_h)),
      ],
      out_specs=[_row_spec(r, d_h), _row_spec(r, 1)],
      out_shape=[
          jax.ShapeDtypeStruct((n, d_h), jnp.float32),
          jax.ShapeDtypeStruct((n, 1), jnp.float32),
      ],
  )


def _make_tc2(n, r, d_h, d_z):
  grid = n // r

  def body(aa_ref, ab_ref, hs1_ref, dinv_ref, b1_ref, w2_ref, hs2_ref):
    dinv = dinv_ref[...]
    h = dinv * (aa_ref[0] + ab_ref[0] + hs1_ref[...]) + b1_ref[...]
    h = jnp.maximum(h, 0.0)
    hs2_ref[...] = dinv * jnp.dot(
        h, w2_ref[...], preferred_element_type=jnp.float32)

  return pl.pallas_call(
      body,
      grid=(grid,),
      in_specs=[
          pl.BlockSpec((1, r, d_h), lambda i: (0, i, 0)),
          pl.BlockSpec((1, r, d_h), lambda i: (1, i, 0)),
          _row_spec(r, d_h),
          _row_spec(r, 1),
          _full_spec((1, d_h)),
          _full_spec((d_h, d_z)),
      ],
      out_specs=[_row_spec(r, d_z)],
      out_shape=[jax.ShapeDtypeStruct((n, d_z), jnp.float32)],
  )


def _make_tc3(n, r, d_z, d_h, d_in):
  grid = n // r

  def body(aa_ref, ab_ref, hs2_ref, dinv_ref, b2_ref, w3_ref, b3_ref, w4_ref,
           b4_ref, xhat_ref, z_ref):
    dinv = dinv_ref[...]
    z = dinv * (aa_ref[0] + ab_ref[0] + hs2_ref[...]) + b2_ref[...]
    z = jnp.maximum(z, 0.0)
    dd = jnp.dot(z, w3_ref[...], preferred_element_type=jnp.float32)
    dd = jnp.maximum(dd + b3_ref[...], 0.0)
    xhat_ref[...] = jnp.dot(
        dd, w4_ref[...], preferred_element_type=jnp.float32) + b4_ref[...]
    z_ref[...] = z

  return pl.pallas_call(
      body,
      grid=(grid,),
      in_specs=[
          pl.BlockSpec((1, r, d_z), lambda i: (0, i, 0)),
          pl.BlockSpec((1, r, d_z), lambda i: (1, i, 0)),
          _row_spec(r, d_z),
          _row_spec(r, 1),
          _full_spec((1, d_z)),
          _full_spec((d_z, d_h)),
          _full_spec((1, d_h)),
          _full_spec((d_h, d_in)),
          _full_spec((1, d_in)),
      ],
      out_specs=[_row_spec(r, d_in), _row_spec(r, d_z)],
      out_shape=[
          jax.ShapeDtypeStruct((n, d_in), jnp.float32),
          jax.ShapeDtypeStruct((n, d_z), jnp.float32),
      ],
  )


@jax.jit
def kernel(x, edge_index, W1, b1, W2, b2, W3, b3, W4, b4):
  n, d_in = x.shape
  d_h = W1.shape[1]
  d_z = W2.shape[1]
  e = edge_index.shape[1]
  assert n + 1 <= N_PAD

  assert e % (NW * CH) == 0
  cpw = e // (NW * CH)
  src3 = edge_index[0].reshape(NW, cpw, CH)
  dst3 = edge_index[1].reshape(NW, cpw, CH)

  deg_parts = _make_deg_kernel(cpw)(dst3)
  dega = deg_parts[0, :n].reshape(n, 1)
  degb = deg_parts[1, :n].reshape(n, 1)

  r = 1000
  hs1, dinv = _make_tc1(n, r, d_in, d_h)(x, dega, degb, W1)

  acc1 = _make_prop_kernel(cpw, d_h)(hs1, src3, dst3)
  hs2, = _make_tc2(n, r, d_h, d_z)(
      acc1, acc1, hs1, dinv, b1.reshape(1, d_h), W2)

  acc2 = _make_prop_kernel(cpw, d_z)(hs2, src3, dst3)
  x_hat, z = _make_tc3(n, r, d_z, d_h, d_in)(
      acc2, acc2, hs2, dinv, b2.reshape(1, d_z), W3,
      b3.reshape(1, d_h), W4, b4.reshape(1, d_in))

  return (x_hat, z)


# speedup vs baseline: 1.5272x; 1.0175x over previous
import functools

import jax
import jax.numpy as jnp
from jax import lax
from jax.experimental import pallas as pl
from jax.experimental.pallas import tpu as pltpu
from jax.experimental.pallas import tpu_sc as plsc

NC = 2
NS = 16
NW = NC * NS
CH = 80
LANES = 16

SPAN = 640
N_PAD = SPAN * NS


def _sc_mesh():
  return plsc.VectorSubcoreMesh(core_axis_name="c", subcore_axis_name="s")


def _make_deg_kernel(cpw):
  @functools.partial(
      pl.kernel,
      out_type=jax.ShapeDtypeStruct((NC, N_PAD), jnp.float32),
      mesh=_sc_mesh(),
      scratch_types=[
          pltpu.VMEM((cpw, CH), jnp.int32),
          pltpu.VMEM((CH,), jnp.float32),
          pltpu.VMEM((CH,), jnp.float32),
          pltpu.VMEM_SHARED((N_PAD,), jnp.float32),
      ],
      compiler_params=pltpu.CompilerParams(use_tc_tiling_on_sc=False),
  )
  def deg_kernel(dst_hbm, out_hbm, dstv, onesv, zerov, acc):
    c = lax.axis_index("c")
    s = lax.axis_index("s")
    w = s * NC + c

    def fill(i, _):
      onesv[pl.ds(i * LANES, LANES)] = jnp.full((LANES,), 1.0, jnp.float32)
      zerov[pl.ds(i * LANES, LANES)] = jnp.zeros((LANES,), jnp.float32)
      return 0

    lax.fori_loop(0, CH // LANES, fill, 0)

    for k in range(SPAN // CH):
      pltpu.sync_copy(zerov, acc.at[pl.ds(s * SPAN + k * CH, CH)])
    plsc.subcore_barrier()

    pltpu.sync_copy(dst_hbm.at[w], dstv)

    def body(j, _):
      pltpu.sync_copy(onesv, acc.at[dstv.at[j]], add=True)
      return 0

    lax.fori_loop(0, cpw, body, 0)
    plsc.subcore_barrier()

    pltpu.sync_copy(acc.at[pl.ds(s * SPAN, SPAN)],
                    out_hbm.at[c, pl.ds(s * SPAN, SPAN)])

  return deg_kernel


def _make_prop_kernel(cpw, d):
  @functools.partial(
      pl.kernel,
      out_type=jax.ShapeDtypeStruct((NC, N_PAD, d), jnp.float32),
      mesh=_sc_mesh(),
      scratch_types=[
          pltpu.VMEM((cpw, CH), jnp.int32),
          pltpu.VMEM((cpw, CH), jnp.int32),
          pltpu.VMEM((CH, d), jnp.float32),
          pltpu.VMEM((CH, d), jnp.float32),
          pltpu.VMEM_SHARED((N_PAD, d), jnp.float32),
          pltpu.SemaphoreType.DMA,
          pltpu.SemaphoreType.DMA,
      ],
      compiler_params=pltpu.CompilerParams(use_tc_tiling_on_sc=False),
  )
  def prop_kernel(tab_hbm, src_hbm, dst_hbm, out_hbm, srcv, dstv, rows0,
                  rows1, acc, sem0, sem1):
    c = lax.axis_index("c")
    s = lax.axis_index("s")
    w = s * NC + c

    cols = d // LANES

    def fill(i, _):
      r = i // cols
      col = (i % cols) * LANES
      rows0[r, pl.ds(col, LANES)] = jnp.zeros((LANES,), jnp.float32)
      return 0

    lax.fori_loop(0, CH * cols, fill, 0)
    for k in range(SPAN // CH):
      pltpu.sync_copy(rows0, acc.at[pl.ds(s * SPAN + k * CH, CH)])
    plsc.subcore_barrier()

    pltpu.sync_copy(src_hbm.at[w], srcv)
    pltpu.sync_copy(dst_hbm.at[w], dstv)

    def body(j, _):
      pltpu.async_copy(tab_hbm.at[srcv.at[j]], rows0, sem0).wait()
      pltpu.sync_copy(rows0, acc.at[dstv.at[j]], add=True)
      return 0

    lax.fori_loop(0, cpw, body, 0)
    plsc.subcore_barrier()

    pltpu.sync_copy(acc.at[pl.ds(s * SPAN, SPAN)],
                    out_hbm.at[c, pl.ds(s * SPAN, SPAN)])

  return prop_kernel


def _row_spec(r, cols):
  return pl.BlockSpec((r, cols), lambda i: (i, 0))


def _full_spec(shape):
  return pl.BlockSpec(shape, lambda i: tuple(0 for _ in shape))


def _make_tc1(n, r, d_in, d_h):
  grid = n // r

  def body(x_ref, dega_ref, degb_ref, w1_ref, hs1_ref, dinv_ref):
    deg = dega_ref[...] + degb_ref[...] + 1.0
    dinv = 1.0 / jnp.sqrt(deg)
    h = jnp.dot(x_ref[...], w1_ref[...], preferred_element_type=jnp.float32)
    hs1_ref[...] = h * dinv
    dinv_ref[...] = dinv

  return pl.pallas_call(
      body,
      grid=(grid,),
      in_specs=[
          _row_spec(r, d_in),
          _row_spec(r, 1),
          _row_spec(r, 1),
          _full_spec((d_in, d_h)),
      ],
      out_specs=[_row_spec(r, d_h), _row_spec(r, 1)],
      out_shape=[
          jax.ShapeDtypeStruct((n, d_h), jnp.float32),
          jax.ShapeDtypeStruct((n, 1), jnp.float32),
      ],
  )


def _make_tc2(n, r, d_h, d_z):
  grid = n // r

  def body(aa_ref, ab_ref, hs1_ref, dinv_ref, b1_ref, w2_ref, hs2_ref):
    dinv = dinv_ref[...]
    h = dinv * (aa_ref[0] + ab_ref[0] + hs1_ref[...]) + b1_ref[...]
    h = jnp.maximum(h, 0.0)
    hs2_ref[...] = dinv * jnp.dot(
        h, w2_ref[...], preferred_element_type=jnp.float32)

  return pl.pallas_call(
      body,
      grid=(grid,),
      in_specs=[
          pl.BlockSpec((1, r, d_h), lambda i: (0, i, 0)),
          pl.BlockSpec((1, r, d_h), lambda i: (1, i, 0)),
          _row_spec(r, d_h),
          _row_spec(r, 1),
          _full_spec((1, d_h)),
          _full_spec((d_h, d_z)),
      ],
      out_specs=[_row_spec(r, d_z)],
      out_shape=[jax.ShapeDtypeStruct((n, d_z), jnp.float32)],
  )


def _make_tc3(n, r, d_z, d_h, d_in):
  grid = n // r

  def body(aa_ref, ab_ref, hs2_ref, dinv_ref, b2_ref, w3_ref, b3_ref, w4_ref,
           b4_ref, xhat_ref, z_ref):
    dinv = dinv_ref[...]
    z = dinv * (aa_ref[0] + ab_ref[0] + hs2_ref[...]) + b2_ref[...]
    z = jnp.maximum(z, 0.0)
    dd = jnp.dot(z, w3_ref[...], preferred_element_type=jnp.float32)
    dd = jnp.maximum(dd + b3_ref[...], 0.0)
    xhat_ref[...] = jnp.dot(
        dd, w4_ref[...], preferred_element_type=jnp.float32) + b4_ref[...]
    z_ref[...] = z

  return pl.pallas_call(
      body,
      grid=(grid,),
      in_specs=[
          pl.BlockSpec((1, r, d_z), lambda i: (0, i, 0)),
          pl.BlockSpec((1, r, d_z), lambda i: (1, i, 0)),
          _row_spec(r, d_z),
          _row_spec(r, 1),
          _full_spec((1, d_z)),
          _full_spec((d_z, d_h)),
          _full_spec((1, d_h)),
          _full_spec((d_h, d_in)),
          _full_spec((1, d_in)),
      ],
      out_specs=[_row_spec(r, d_in), _row_spec(r, d_z)],
      out_shape=[
          jax.ShapeDtypeStruct((n, d_in), jnp.float32),
          jax.ShapeDtypeStruct((n, d_z), jnp.float32),
      ],
  )


@jax.jit
def kernel(x, edge_index, W1, b1, W2, b2, W3, b3, W4, b4):
  n, d_in = x.shape
  d_h = W1.shape[1]
  d_z = W2.shape[1]
  e = edge_index.shape[1]
  assert n + 1 <= N_PAD

  assert e % (NW * CH) == 0
  cpw = e // (NW * CH)
  src3 = edge_index[0].reshape(NW, cpw, CH)
  dst3 = edge_index[1].reshape(NW, cpw, CH)

  deg_parts = _make_deg_kernel(cpw)(dst3)
  dega = deg_parts[0, :n].reshape(n, 1)
  degb = deg_parts[1, :n].reshape(n, 1)

  r = 2000
  hs1, dinv = _make_tc1(n, r, d_in, d_h)(x, dega, degb, W1)

  acc1 = _make_prop_kernel(cpw, d_h)(hs1, src3, dst3)
  hs2, = _make_tc2(n, r, d_h, d_z)(
      acc1, acc1, hs1, dinv, b1.reshape(1, d_h), W2)

  acc2 = _make_prop_kernel(cpw, d_z)(hs2, src3, dst3)
  x_hat, z = _make_tc3(n, r, d_z, d_h, d_in)(
      acc2, acc2, hs2, dinv, b2.reshape(1, d_z), W3,
      b3.reshape(1, d_h), W4, b4.reshape(1, d_in))

  return (x_hat, z)
